# baseline (speedup 1.0000x reference)
import functools

import jax
import jax.numpy as jnp
from jax.experimental import pallas as pl
from jax.experimental.pallas import tpu as pltpu

_B, _N, _C = 32, 8732, 21


def _isd_body(lam_ref, conf_ref, shuf_ref, interp_ref, out_ref, acc_ref):
    b = pl.program_id(0)

    x = conf_ref[0]
    t = shuf_ref[0]
    ci = interp_ref[0]
    lam = lam_ref[0]

    mixed = lam * x + (1.0 - lam) * t + 1e-7
    interp = ci + 1e-7

    lmax = jnp.max(x[:, 1:], axis=1)
    rmax = jnp.max(t[:, 1:], axis=1)
    mask = jnp.logical_and(lmax > x[:, 0], rmax > t[:, 0])
    maskf = mask.astype(jnp.float32)

    d = interp - mixed
    ld = jnp.log(interp / mixed)
    klrow = jnp.sum(d * ld, axis=1)

    s = jnp.sum(klrow * maskf)
    c = jnp.sum(maskf)

    @pl.when(b == 0)
    def _():
        acc_ref[0] = 0.0
        acc_ref[1] = 0.0

    acc_ref[0] += s
    acc_ref[1] += c

    @pl.when(b == pl.num_programs(0) - 1)
    def _():
        tot = acc_ref[0]
        cnt = acc_ref[1]
        loss = 0.5 * tot / jnp.maximum(cnt, 1.0)
        out_ref[0] = jnp.where(cnt > 0.0, loss, 0.0)


@functools.partial(jax.jit, static_argnames=("interpret",))
def _isd_loss_tc(lam, conf, conf_shuffle, conf_interpolation, interpret=False):
    half = _B // 2
    grid = (_B,)
    out = pl.pallas_call(
        _isd_body,
        grid=grid,
        in_specs=[
            pl.BlockSpec(memory_space=pltpu.SMEM),
            pl.BlockSpec((1, _N, _C), lambda b: (b, 0, 0)),
            pl.BlockSpec((1, _N, _C), lambda b: ((b + half) % _B, 0, 0)),
            pl.BlockSpec((1, _N, _C), lambda b: (b, 0, 0)),
        ],
        out_specs=pl.BlockSpec(memory_space=pltpu.SMEM),
        out_shape=jax.ShapeDtypeStruct((1,), jnp.float32),
        scratch_shapes=[pltpu.SMEM((2,), jnp.float32)],
        compiler_params=pltpu.CompilerParams(
            dimension_semantics=("arbitrary",),
        ),
        interpret=interpret,
    )(jnp.asarray(lam, jnp.float32).reshape(1), conf, conf_shuffle,
      conf_interpolation)
    return out[0]


def kernel(lam, conf, conf_flip, loc, loc_flip, conf_shuffle,
           conf_interpolation, loc_shuffle, loc_interpolation):
    return _isd_loss_tc(lam, conf, conf_shuffle, conf_interpolation)
